# baseline (speedup 1.0000x reference)
import jax
import jax.numpy as jnp
from jax import lax
from jax.experimental import pallas as pl
from jax.experimental.pallas import tpu as pltpu
from jax.experimental.pallas import tpu_sc as plsc

_B = 4
_NS = 2048
_NR = 8192

_TC_NR = 7168
_BLK = 1024
_NJ = _TC_NR // _BLK

_INFO = plsc.get_sparse_core_info()
_NWORK = _INFO.num_cores * _INFO.num_subcores
_SC_NR = _NR - _TC_NR
_RPW = _SC_NR // _NWORK
_NRV = _RPW // 16
_NSV = _NS // 16
_SJB = 8
_NBT = _NSV // _SJB
_INF = 3.4e38


def _tc_kernel(s_ref, r_ref, lb_ref, samp_ref):
    b = pl.program_id(0)
    j = pl.program_id(1)

    @pl.when(jnp.logical_and(b == 0, j == 0))
    def _init_out():
        lb_ref[0, 0] = 0.0

    s = s_ref[0]
    rb = r_ref[0]
    r2 = jnp.sum(rb * rb, axis=1, keepdims=True)
    s2 = jnp.sum(s * s, axis=0, keepdims=True)
    sxm2 = s[0:1, :] * -2.0
    sym2 = s[1:2, :] * -2.0
    szm2 = s[2:3, :] * -2.0
    rx = rb[:, 0:1]
    ry = rb[:, 1:2]
    rz = rb[:, 2:3]
    d2 = r2 + s2
    d2 = d2 + rx * sxm2
    d2 = d2 + ry * sym2
    d2 = d2 + rz * szm2

    raw_min = jnp.min(d2, axis=1, keepdims=True)
    raw_sum = jnp.sum(jnp.sqrt(jnp.maximum(raw_min, 0.0)))
    lb_ref[0, 0] += raw_sum * (5.0 / (_B * _NR))

    samp_min = jnp.min(d2, axis=0, keepdims=True)

    @pl.when(j == 0)
    def _init_acc():
        samp_ref[0] = samp_min

    @pl.when(j != 0)
    def _acc():
        samp_ref[0] = jnp.minimum(samp_ref[0], samp_min)


def _sc_body(s3_hbm, r3_hbm, part_hbm, rawmin_hbm,
             sx_v, sy_v, sz_v, sxm2_v, sym2_v, szm2_v, s2_v,
             rx_v, ry_v, rz_v, r2_v, rowmin_v, cm_v):
    wid = lax.axis_index("s") * _INFO.num_cores + lax.axis_index("c")
    lane = lax.iota(jnp.int32, 16)

    for b in range(_B):
        pltpu.sync_copy(s3_hbm.at[pl.ds((b * 3 + 0) * _NS, _NS)], sx_v)
        pltpu.sync_copy(s3_hbm.at[pl.ds((b * 3 + 1) * _NS, _NS)], sy_v)
        pltpu.sync_copy(s3_hbm.at[pl.ds((b * 3 + 2) * _NS, _NS)], sz_v)
        roff = _TC_NR + wid * _RPW
        pltpu.sync_copy(r3_hbm.at[pl.ds((b * 3 + 0) * _NR + roff, _RPW)], rx_v)
        pltpu.sync_copy(r3_hbm.at[pl.ds((b * 3 + 1) * _NR + roff, _RPW)], ry_v)
        pltpu.sync_copy(r3_hbm.at[pl.ds((b * 3 + 2) * _NR + roff, _RPW)], rz_v)

        def _prep_s(i, _):
            o = i * 16
            sx = sx_v[pl.ds(o, 16)]
            sy = sy_v[pl.ds(o, 16)]
            sz = sz_v[pl.ds(o, 16)]
            sxm2_v[pl.ds(o, 16)] = sx * -2.0
            sym2_v[pl.ds(o, 16)] = sy * -2.0
            szm2_v[pl.ds(o, 16)] = sz * -2.0
            s2_v[pl.ds(o, 16)] = sx * sx + sy * sy + sz * sz
            return 0

        lax.fori_loop(0, _NSV, _prep_s, 0)

        def _prep_r(i, _):
            o = i * 16
            rx = rx_v[pl.ds(o, 16)]
            ry = ry_v[pl.ds(o, 16)]
            rz = rz_v[pl.ds(o, 16)]
            r2_v[pl.ds(o, 16)] = rx * rx + ry * ry + rz * rz
            cm_v[pl.ds(o, 16)] = jnp.full((16,), _INF)
            return 0

        lax.fori_loop(0, _NRV, _prep_r, 0)

        def _batch(bt, _):
            sboff = bt * (_SJB * 16)
            svx = [sxm2_v[pl.ds(sboff + sj * 16, 16)] for sj in range(_SJB)]
            svy = [sym2_v[pl.ds(sboff + sj * 16, 16)] for sj in range(_SJB)]
            svz = [szm2_v[pl.ds(sboff + sj * 16, 16)] for sj in range(_SJB)]
            sv2 = [s2_v[pl.ds(sboff + sj * 16, 16)] for sj in range(_SJB)]

            def _rj(rj, rowacc):
                base = rj * 16
                colvec = jnp.full((16,), _INF)
                new = list(rowacc)
                for l in range(16):
                    idx = jnp.full((16,), base + l, jnp.int32)
                    rxs = plsc.load_gather(rx_v, [idx])
                    rys = plsc.load_gather(ry_v, [idx])
                    rzs = plsc.load_gather(rz_v, [idx])
                    r2s = plsc.load_gather(r2_v, [idx])
                    dcol = jnp.full((16,), _INF)
                    for sj in range(_SJB):
                        d = (sv2[sj] + r2s) + (
                            svx[sj] * rxs + svy[sj] * rys + svz[sj] * rzs)
                        new[sj] = jnp.minimum(new[sj], d)
                        dcol = jnp.minimum(dcol, d)
                    colvec = jnp.where(lane == l, jnp.min(dcol), colvec)
                cmc = cm_v[pl.ds(base, 16)]
                cm_v[pl.ds(base, 16)] = jnp.minimum(cmc, colvec)
                return tuple(new)

            rowacc = lax.fori_loop(
                0, _NRV, _rj,
                tuple(jnp.full((16,), _INF) for _ in range(_SJB)))
            for sj in range(_SJB):
                rowmin_v[pl.ds(sboff + sj * 16, 16)] = rowacc[sj]
            return 0

        lax.fori_loop(0, _NBT, _batch, 0)

        pltpu.sync_copy(rowmin_v,
                        part_hbm.at[pl.ds((b * _NWORK + wid) * _NS, _NS)])
        pltpu.sync_copy(cm_v,
                        rawmin_hbm.at[pl.ds(b * _SC_NR + wid * _RPW, _RPW)])


def _combine_kernel(lb_ref, tc_samp_ref, part_ref, rawmin_ref, out_ref):
    samp = jnp.minimum(
        tc_samp_ref[:, 0, :], jnp.min(part_ref[...], axis=1))
    sampd = jnp.sqrt(jnp.maximum(samp, 0.0))
    rawd = jnp.sqrt(jnp.maximum(rawmin_ref[...], 0.0))
    lb_sc = jnp.sum(rawd) * (5.0 / (_B * _NR))
    per_cloud = (
        jnp.mean(sampd, axis=1, keepdims=True)
        + jnp.max(sampd, axis=1, keepdims=True))
    out_ref[0, 0] = lb_ref[0, 0] + lb_sc + jnp.sum(per_cloud) * (1.0 / _B)


@jax.jit
def kernel(sampled_lidar_list, raw_lidar_list):
    s3 = jnp.transpose(sampled_lidar_list[:, :, 0:3], (0, 2, 1))
    r = raw_lidar_list[:, :, 0:3]
    r3_flat = jnp.transpose(r, (0, 2, 1)).reshape(-1)
    s3_flat = s3.reshape(-1)

    sc = pl.kernel(
        _sc_body,
        out_type=[
            jax.ShapeDtypeStruct((_B * _NWORK * _NS,), jnp.float32),
            jax.ShapeDtypeStruct((_B * _SC_NR,), jnp.float32),
        ],
        mesh=plsc.VectorSubcoreMesh(core_axis_name="c", subcore_axis_name="s"),
        compiler_params=pltpu.CompilerParams(needs_layout_passes=False),
        scratch_types=[
            pltpu.VMEM((_NS,), jnp.float32),
            pltpu.VMEM((_NS,), jnp.float32),
            pltpu.VMEM((_NS,), jnp.float32),
            pltpu.VMEM((_NS,), jnp.float32),
            pltpu.VMEM((_NS,), jnp.float32),
            pltpu.VMEM((_NS,), jnp.float32),
            pltpu.VMEM((_NS,), jnp.float32),
            pltpu.VMEM((_RPW,), jnp.float32),
            pltpu.VMEM((_RPW,), jnp.float32),
            pltpu.VMEM((_RPW,), jnp.float32),
            pltpu.VMEM((_RPW,), jnp.float32),
            pltpu.VMEM((_NS,), jnp.float32),
            pltpu.VMEM((_RPW,), jnp.float32),
        ],
    )
    part, sc_rawmin = sc(s3_flat, r3_flat)
    part = part.reshape(_B, _NWORK, _NS)
    sc_rawmin = sc_rawmin.reshape(_B, _SC_NR)

    lb, tc_samp = pl.pallas_call(
        _tc_kernel,
        grid=(_B, _NJ),
        in_specs=[
            pl.BlockSpec((1, 3, _NS), lambda b, j: (b, 0, 0)),
            pl.BlockSpec((1, _BLK, 3), lambda b, j: (b, j, 0)),
        ],
        out_specs=[
            pl.BlockSpec((1, 1), lambda b, j: (0, 0),
                         memory_space=pltpu.SMEM),
            pl.BlockSpec((1, 1, _NS), lambda b, j: (b, 0, 0)),
        ],
        out_shape=[
            jax.ShapeDtypeStruct((1, 1), jnp.float32),
            jax.ShapeDtypeStruct((_B, 1, _NS), jnp.float32),
        ],
    )(s3, r)

    out = pl.pallas_call(
        _combine_kernel,
        out_specs=pl.BlockSpec(memory_space=pltpu.SMEM),
        out_shape=jax.ShapeDtypeStruct((1, 1), jnp.float32),
    )(lb, tc_samp, part, sc_rawmin)
    return out[0, 0]


# speedup vs baseline: 1.1870x; 1.1870x over previous
import jax
import jax.numpy as jnp
from jax import lax
from jax.experimental import pallas as pl
from jax.experimental.pallas import tpu as pltpu
from jax.experimental.pallas import tpu_sc as plsc

_B = 4
_NS = 2048
_NR = 8192

_TC_NR = 6144
_BLK = 2048
_NJ = _TC_NR // _BLK

_INFO = plsc.get_sparse_core_info()
_NWORK = _INFO.num_cores * _INFO.num_subcores
_SC_NR = _NR - _TC_NR
_RPW = _SC_NR // _NWORK
_NRV = _RPW // 16
_NSV = _NS // 16
_SJB = 8
_NBT = _NSV // _SJB
_INF = 3.4e38


def _tc_kernel(s_ref, r_ref, lb_ref, samp_ref):
    b = pl.program_id(0)
    j = pl.program_id(1)

    @pl.when(jnp.logical_and(b == 0, j == 0))
    def _init_out():
        lb_ref[0, 0] = 0.0

    s = s_ref[0]
    rb = r_ref[0]
    r2 = jnp.sum(rb * rb, axis=1, keepdims=True)
    s2 = jnp.sum(s * s, axis=0, keepdims=True)
    sxm2 = s[0:1, :] * -2.0
    sym2 = s[1:2, :] * -2.0
    szm2 = s[2:3, :] * -2.0
    rx = rb[:, 0:1]
    ry = rb[:, 1:2]
    rz = rb[:, 2:3]
    d2 = r2 + s2
    d2 = d2 + rx * sxm2
    d2 = d2 + ry * sym2
    d2 = d2 + rz * szm2

    raw_min = jnp.min(d2, axis=1, keepdims=True)
    raw_sum = jnp.sum(jnp.sqrt(jnp.maximum(raw_min, 0.0)))
    lb_ref[0, 0] += raw_sum * (5.0 / (_B * _NR))

    samp_min = jnp.min(d2, axis=0, keepdims=True)

    @pl.when(j == 0)
    def _init_acc():
        samp_ref[0] = samp_min

    @pl.when(j != 0)
    def _acc():
        samp_ref[0] = jnp.minimum(samp_ref[0], samp_min)


def _sc_body(s3_hbm, rsc_hbm, part_hbm, rawmin_hbm,
             s_all_v, sxm2_v, sym2_v, szm2_v, s2_v,
             r_all_v, r2_v, rowmin_v, cm_v):
    wid = lax.axis_index("s") * _INFO.num_cores + lax.axis_index("c")
    lane = lax.iota(jnp.int32, 16)

    pltpu.sync_copy(s3_hbm, s_all_v)
    pltpu.sync_copy(rsc_hbm.at[pl.ds(wid * (_B * 3 * _RPW), _B * 3 * _RPW)],
                    r_all_v)

    for b in range(_B):
        sx_o = (b * 3 + 0) * _NS
        sy_o = (b * 3 + 1) * _NS
        sz_o = (b * 3 + 2) * _NS
        rx_o = (b * 3 + 0) * _RPW
        ry_o = (b * 3 + 1) * _RPW
        rz_o = (b * 3 + 2) * _RPW

        def _prep_s(i, _):
            o = i * 16
            sx = s_all_v[pl.ds(sx_o + o, 16)]
            sy = s_all_v[pl.ds(sy_o + o, 16)]
            sz = s_all_v[pl.ds(sz_o + o, 16)]
            sxm2_v[pl.ds(o, 16)] = sx * -2.0
            sym2_v[pl.ds(o, 16)] = sy * -2.0
            szm2_v[pl.ds(o, 16)] = sz * -2.0
            s2_v[pl.ds(o, 16)] = sx * sx + sy * sy + sz * sz
            return 0

        lax.fori_loop(0, _NSV, _prep_s, 0)

        def _prep_r(i, _):
            o = i * 16
            rx = r_all_v[pl.ds(rx_o + o, 16)]
            ry = r_all_v[pl.ds(ry_o + o, 16)]
            rz = r_all_v[pl.ds(rz_o + o, 16)]
            r2_v[pl.ds(o, 16)] = rx * rx + ry * ry + rz * rz
            cm_v[pl.ds(b * _RPW + o, 16)] = jnp.full((16,), _INF)
            return 0

        lax.fori_loop(0, _NRV, _prep_r, 0)

        def _batch(bt, _):
            sboff = bt * (_SJB * 16)
            svx = [sxm2_v[pl.ds(sboff + sj * 16, 16)] for sj in range(_SJB)]
            svy = [sym2_v[pl.ds(sboff + sj * 16, 16)] for sj in range(_SJB)]
            svz = [szm2_v[pl.ds(sboff + sj * 16, 16)] for sj in range(_SJB)]
            sv2 = [s2_v[pl.ds(sboff + sj * 16, 16)] for sj in range(_SJB)]

            def _rj(rj, rowacc):
                base = rj * 16
                colvec = jnp.full((16,), _INF)
                new = list(rowacc)
                for l in range(16):
                    idx = jnp.full((16,), base + l, jnp.int32)
                    rxs = plsc.load_gather(r_all_v, [idx + rx_o])
                    rys = plsc.load_gather(r_all_v, [idx + ry_o])
                    rzs = plsc.load_gather(r_all_v, [idx + rz_o])
                    r2s = plsc.load_gather(r2_v, [idx])
                    dcol = jnp.full((16,), _INF)
                    for sj in range(_SJB):
                        d = (sv2[sj] + r2s) + (
                            svx[sj] * rxs + svy[sj] * rys + svz[sj] * rzs)
                        new[sj] = jnp.minimum(new[sj], d)
                        dcol = jnp.minimum(dcol, d)
                    colvec = jnp.where(lane == l, jnp.min(dcol), colvec)
                cmc = cm_v[pl.ds(b * _RPW + base, 16)]
                cm_v[pl.ds(b * _RPW + base, 16)] = jnp.minimum(cmc, colvec)
                return tuple(new)

            rowacc = lax.fori_loop(
                0, _NRV, _rj,
                tuple(jnp.full((16,), _INF) for _ in range(_SJB)))
            for sj in range(_SJB):
                rowmin_v[pl.ds(b * _NS + sboff + sj * 16, 16)] = rowacc[sj]
            return 0

        lax.fori_loop(0, _NBT, _batch, 0)

    pltpu.sync_copy(rowmin_v, part_hbm.at[pl.ds(wid * (_B * _NS), _B * _NS)])
    pltpu.sync_copy(cm_v, rawmin_hbm.at[pl.ds(wid * (_B * _RPW), _B * _RPW)])


def _combine_kernel(lb_ref, tc_samp_ref, part_ref, rawmin_ref, out_ref):
    samp = jnp.minimum(
        tc_samp_ref[:, 0, :], jnp.min(part_ref[...], axis=0))
    sampd = jnp.sqrt(jnp.maximum(samp, 0.0))
    rawd = jnp.sqrt(jnp.maximum(rawmin_ref[...], 0.0))
    lb_sc = jnp.sum(rawd) * (5.0 / (_B * _NR))
    per_cloud = (
        jnp.mean(sampd, axis=1, keepdims=True)
        + jnp.max(sampd, axis=1, keepdims=True))
    out_ref[0, 0] = lb_ref[0, 0] + lb_sc + jnp.sum(per_cloud) * (1.0 / _B)


@jax.jit
def kernel(sampled_lidar_list, raw_lidar_list):
    s3 = jnp.transpose(sampled_lidar_list[:, :, 0:3], (0, 2, 1))
    r = raw_lidar_list[:, :, 0:3]
    rsc = jnp.transpose(r[:, _TC_NR:, :], (0, 2, 1))
    rsc = rsc.reshape(_B, 3, _NWORK, _RPW)
    rsc_flat = jnp.transpose(rsc, (2, 0, 1, 3)).reshape(-1)
    s3_flat = s3.reshape(-1)

    sc = pl.kernel(
        _sc_body,
        out_type=[
            jax.ShapeDtypeStruct((_NWORK * _B * _NS,), jnp.float32),
            jax.ShapeDtypeStruct((_NWORK * _B * _RPW,), jnp.float32),
        ],
        mesh=plsc.VectorSubcoreMesh(core_axis_name="c", subcore_axis_name="s"),
        compiler_params=pltpu.CompilerParams(needs_layout_passes=False),
        scratch_types=[
            pltpu.VMEM((_B * 3 * _NS,), jnp.float32),
            pltpu.VMEM((_NS,), jnp.float32),
            pltpu.VMEM((_NS,), jnp.float32),
            pltpu.VMEM((_NS,), jnp.float32),
            pltpu.VMEM((_NS,), jnp.float32),
            pltpu.VMEM((_B * 3 * _RPW,), jnp.float32),
            pltpu.VMEM((_RPW,), jnp.float32),
            pltpu.VMEM((_B * _NS,), jnp.float32),
            pltpu.VMEM((_B * _RPW,), jnp.float32),
        ],
    )
    part, sc_rawmin = sc(s3_flat, rsc_flat)
    part = part.reshape(_NWORK, _B, _NS)
    sc_rawmin = sc_rawmin.reshape(_NWORK * _B, _RPW)

    lb, tc_samp = pl.pallas_call(
        _tc_kernel,
        grid=(_B, _NJ),
        in_specs=[
            pl.BlockSpec((1, 3, _NS), lambda b, j: (b, 0, 0)),
            pl.BlockSpec((1, _BLK, 3), lambda b, j: (b, j, 0)),
        ],
        out_specs=[
            pl.BlockSpec((1, 1), lambda b, j: (0, 0),
                         memory_space=pltpu.SMEM),
            pl.BlockSpec((1, 1, _NS), lambda b, j: (b, 0, 0)),
        ],
        out_shape=[
            jax.ShapeDtypeStruct((1, 1), jnp.float32),
            jax.ShapeDtypeStruct((_B, 1, _NS), jnp.float32),
        ],
    )(s3, r)

    out = pl.pallas_call(
        _combine_kernel,
        out_specs=pl.BlockSpec(memory_space=pltpu.SMEM),
        out_shape=jax.ShapeDtypeStruct((1, 1), jnp.float32),
    )(lb, tc_samp, part, sc_rawmin)
    return out[0, 0]
